# baseline (speedup 1.0000x reference)
import functools

import jax
import jax.numpy as jnp
from jax import lax
from jax.experimental import pallas as pl
from jax.experimental.pallas import tpu as pltpu
from jax.experimental.pallas import tpu_sc as plsc

N_EDGES = 320000
D = 128
N_NODES = 10000
N_PAD = 10240
NC = 2
NS = 16
L = 16
NW = NC * NS
EPT = N_EDGES // NW
B = 80
NCHUNK = EPT // B
NBUF = 4
RPT = N_PAD // NS
R2 = N_PAD // NW

_mesh = plsc.VectorSubcoreMesh(core_axis_name="c", subcore_axis_name="s")


def _zero_vmem(ref, nwords):
    def body(j, _):
        ref[pl.ds(j * L, L)] = jnp.zeros((L,), jnp.float32)
        return 0
    lax.fori_loop(0, nwords // L, body, 0)


@functools.partial(
    pl.kernel,
    out_type=(
        jax.ShapeDtypeStruct((NC, N_PAD, D), jnp.float32),
        jax.ShapeDtypeStruct((NC * N_PAD,), jnp.float32),
    ),
    mesh=_mesh,
    scratch_types=[
        pltpu.VMEM_SHARED((N_PAD, D), jnp.float32),
        pltpu.VMEM_SHARED((N_PAD,), jnp.float32),
        pltpu.VMEM((NBUF, B), jnp.int32),
        pltpu.VMEM((B, D), jnp.float32),
        pltpu.VMEM((B, D), jnp.float32),
        pltpu.VMEM((B, D), jnp.float32),
        pltpu.VMEM((B, D), jnp.float32),
        pltpu.VMEM((B,), jnp.float32),
        pltpu.VMEM((RPT,), jnp.float32),
        pltpu.SemaphoreType.DMA,
        pltpu.SemaphoreType.DMA,
        pltpu.SemaphoreType.DMA,
        pltpu.SemaphoreType.DMA,
        pltpu.SemaphoreType.DMA,
        pltpu.SemaphoreType.DMA,
    ],
)
def _scatter_stage(msg_hbm, idx_hbm, psum_hbm, pcnt_hbm,
                   acc_sh, cnt_sh, idx_ring, rows0, rows1, rows2, rows3,
                   ones_v, zvec_v, fsem0, fsem1, fsem2, fsem3, ssem, csem):
    cid = lax.axis_index("c")
    sid = lax.axis_index("s")
    wid = cid * NS + sid
    ebase = wid * EPT
    bufs = (rows0, rows1, rows2, rows3)
    fsems = (fsem0, fsem1, fsem2, fsem3)

    def zrow(r, _):
        def zcol(j, _):
            rows0[r, pl.ds(j * L, L)] = jnp.zeros((L,), jnp.float32)
            return 0
        lax.fori_loop(0, D // L, zcol, 0)
        return 0
    lax.fori_loop(0, B, zrow, 0)
    _zero_vmem(zvec_v, RPT)

    def one(j, _):
        ones_v[pl.ds(j * L, L)] = jnp.ones((L,), jnp.float32)
        return 0
    lax.fori_loop(0, B // L, one, 0)

    base_r = sid * RPT
    for k in range(RPT // B):
        pltpu.sync_copy(rows0, acc_sh.at[pl.ds(base_r + k * B, B), :])
    pltpu.sync_copy(zvec_v, cnt_sh.at[pl.ds(base_r, RPT)])
    plsc.subcore_barrier()

    def fetch_start(c, k):
        pltpu.async_copy(msg_hbm.at[pl.ds(ebase + c * B, B), :],
                         bufs[k], fsems[k])
        pltpu.async_copy(idx_hbm.at[pl.ds(ebase + c * B, B)],
                         idx_ring.at[k], fsems[k])

    def fetch_wait(c, k):
        pltpu.make_async_copy(msg_hbm.at[pl.ds(ebase + c * B, B), :],
                              bufs[k], fsems[k]).wait()
        pltpu.make_async_copy(idx_hbm.at[pl.ds(ebase + c * B, B)],
                              idx_ring.at[k], fsems[k]).wait()

    def scat(c, k):
        pltpu.async_copy(bufs[k], acc_sh.at[idx_ring.at[k]], ssem, add=True)
        pltpu.async_copy(ones_v, cnt_sh.at[idx_ring.at[k]], csem, add=True)
        pltpu.make_async_copy(bufs[k], acc_sh.at[idx_ring.at[k]], ssem).wait()

    for k in range(NBUF):
        fetch_start(k, k)

    def quad(g, _):
        for k in range(NBUF):
            c = NBUF * g + k
            fetch_wait(c, k)
            scat(c, k)

            def refill(c=c, k=k):
                fetch_start(c + NBUF, k)
            pl.when(c + NBUF <= NCHUNK - 1)(refill)
        return 0
    lax.fori_loop(0, (NCHUNK - 1) // NBUF, quad, 0)

    fetch_wait(NCHUNK - 1, 0)
    scat(NCHUNK - 1, 0)

    def drain(i, _):
        pltpu.make_async_copy(ones_v, cnt_sh.at[idx_ring.at[0]], csem).wait()
        return 0
    lax.fori_loop(0, NCHUNK, drain, 0)
    plsc.subcore_barrier()

    pltpu.sync_copy(acc_sh.at[pl.ds(base_r, RPT), :],
                    psum_hbm.at[cid, pl.ds(base_r, RPT), :])
    pltpu.sync_copy(cnt_sh.at[pl.ds(base_r, RPT)],
                    pcnt_hbm.at[pl.ds(cid * N_PAD + base_r, RPT)])


@functools.partial(
    pl.kernel,
    out_type=jax.ShapeDtypeStruct((N_NODES, D), jnp.float32),
    mesh=_mesh,
    scratch_types=[
        pltpu.VMEM((R2, D), jnp.float32),
        pltpu.VMEM((R2, D), jnp.float32),
        pltpu.VMEM((R2,), jnp.float32),
        pltpu.VMEM((R2,), jnp.float32),
        pltpu.SemaphoreType.DMA,
        pltpu.SemaphoreType.DMA,
    ],
)
def _combine_stage(psum_hbm, pcnt_hbm, out_hbm, pa, pb, ca, cb, ssem, csem):
    cid = lax.axis_index("c")
    sid = lax.axis_index("s")
    wid = cid * NS + sid
    base = wid * R2

    cp_a = pltpu.async_copy(psum_hbm.at[0, pl.ds(base, R2), :], pa, ssem)
    cp_b = pltpu.async_copy(psum_hbm.at[1, pl.ds(base, R2), :], pb, ssem)
    cp_c = pltpu.async_copy(pcnt_hbm.at[pl.ds(base, R2)], ca, csem)
    cp_d = pltpu.async_copy(pcnt_hbm.at[pl.ds(N_PAD + base, R2)], cb, csem)
    cp_c.wait()
    cp_d.wait()
    cp_a.wait()
    cp_b.wait()

    def group(g, _):
        cv = ca[pl.ds(g * L, L)] + cb[pl.ds(g * L, L)]
        rv = 1.0 / jnp.maximum(cv, 1.0)
        for r16 in range(L):
            r = g * L + r16
            s = rv[r16]
            for j in range(D // L):
                pa[r, pl.ds(j * L, L)] = (
                    pa[r, pl.ds(j * L, L)] + pb[r, pl.ds(j * L, L)]) * s
        return 0
    lax.fori_loop(0, R2 // L, group, 0)

    @pl.when(wid < NW - 1)
    def _():
        pltpu.sync_copy(pa, out_hbm.at[pl.ds(base, R2), :])

    @pl.when(wid == NW - 1)
    def _():
        pltpu.sync_copy(pa.at[pl.ds(0, N_NODES - (NW - 1) * R2), :],
                        out_hbm.at[pl.ds((NW - 1) * R2,
                                         N_NODES - (NW - 1) * R2), :])


def kernel(msg, index, t, dim_size):
    del t, dim_size
    idx32 = index.astype(jnp.int32)
    psum, pcnt = _scatter_stage(msg, idx32)
    return _combine_stage(psum, pcnt)


# speedup vs baseline: 1.0159x; 1.0159x over previous
import functools

import jax
import jax.numpy as jnp
from jax import lax
from jax.experimental import pallas as pl
from jax.experimental.pallas import tpu as pltpu
from jax.experimental.pallas import tpu_sc as plsc

N_EDGES = 320000
D = 128
N_NODES = 10000
N_PAD = 10240
NC = 2
NS = 16
L = 16
NW = NC * NS
EPT = N_EDGES // NW
B = 80
NCHUNK = EPT // B
NBUF = 4
RPT = N_PAD // NS
R2 = N_PAD // NW
NBLK = R2 // B

_mesh = plsc.VectorSubcoreMesh(core_axis_name="c", subcore_axis_name="s")


def _zero_vmem(ref, nwords):
    def body(j, _):
        ref[pl.ds(j * L, L)] = jnp.zeros((L,), jnp.float32)
        return 0
    lax.fori_loop(0, nwords // L, body, 0)


@functools.partial(
    pl.kernel,
    out_type=(
        jax.ShapeDtypeStruct((N_NODES, D), jnp.float32),
        jax.ShapeDtypeStruct((NC, N_PAD, D), jnp.float32),
        jax.ShapeDtypeStruct((NC * N_PAD,), jnp.float32),
        jax.ShapeDtypeStruct((NC * L,), jnp.float32),
    ),
    mesh=_mesh,
    scratch_types=[
        pltpu.VMEM_SHARED((N_PAD, D), jnp.float32),
        pltpu.VMEM_SHARED((N_PAD,), jnp.float32),
        pltpu.VMEM((NBUF, B), jnp.int32),
        pltpu.VMEM((B, D), jnp.float32),
        pltpu.VMEM((B, D), jnp.float32),
        pltpu.VMEM((B, D), jnp.float32),
        pltpu.VMEM((B, D), jnp.float32),
        pltpu.VMEM((B,), jnp.float32),
        pltpu.VMEM((RPT,), jnp.float32),
        pltpu.VMEM((R2,), jnp.float32),
        pltpu.VMEM((R2,), jnp.float32),
        pltpu.VMEM((B + L,), jnp.float32),
        pltpu.VMEM((L,), jnp.float32),
        pltpu.SemaphoreType.DMA,
        pltpu.SemaphoreType.DMA,
        pltpu.SemaphoreType.DMA,
        pltpu.SemaphoreType.DMA,
        pltpu.SemaphoreType.DMA,
        pltpu.SemaphoreType.DMA,
    ],
)
def _fused_stage(msg_hbm, idx_hbm, out_hbm, psum_hbm, pcnt_hbm, flag_hbm,
                 acc_sh, cnt_sh, idx_ring, rows0, rows1, rows2, rows3,
                 ones_v, zvec_v, ca, cb, rcpb, fbuf,
                 fsem0, fsem1, fsem2, fsem3, ssem, csem):
    cid = lax.axis_index("c")
    sid = lax.axis_index("s")
    wid = cid * NS + sid
    ebase = wid * EPT
    bufs = (rows0, rows1, rows2, rows3)
    fsems = (fsem0, fsem1, fsem2, fsem3)

    def zrow(r, _):
        def zcol(j, _):
            rows0[r, pl.ds(j * L, L)] = jnp.zeros((L,), jnp.float32)
            return 0
        lax.fori_loop(0, D // L, zcol, 0)
        return 0
    lax.fori_loop(0, B, zrow, 0)
    _zero_vmem(zvec_v, RPT)

    def one(j, _):
        ones_v[pl.ds(j * L, L)] = jnp.ones((L,), jnp.float32)
        return 0
    lax.fori_loop(0, B // L, one, 0)

    @pl.when(sid == 0)
    def _():
        pltpu.sync_copy(zvec_v.at[pl.ds(0, L)], flag_hbm.at[pl.ds(cid * L, L)])

    base_r = sid * RPT
    for k in range(RPT // B):
        pltpu.sync_copy(rows0, acc_sh.at[pl.ds(base_r + k * B, B), :])
    pltpu.sync_copy(zvec_v, cnt_sh.at[pl.ds(base_r, RPT)])
    plsc.subcore_barrier()

    def fetch_start(c, k):
        pltpu.async_copy(msg_hbm.at[pl.ds(ebase + c * B, B), :],
                         bufs[k], fsems[k])
        pltpu.async_copy(idx_hbm.at[pl.ds(ebase + c * B, B)],
                         idx_ring.at[k], fsems[k])

    def fetch_wait(c, k):
        pltpu.make_async_copy(msg_hbm.at[pl.ds(ebase + c * B, B), :],
                              bufs[k], fsems[k]).wait()
        pltpu.make_async_copy(idx_hbm.at[pl.ds(ebase + c * B, B)],
                              idx_ring.at[k], fsems[k]).wait()

    def scat(c, k):
        pltpu.async_copy(bufs[k], acc_sh.at[idx_ring.at[k]], ssem, add=True)
        pltpu.async_copy(ones_v, cnt_sh.at[idx_ring.at[k]], csem, add=True)
        pltpu.make_async_copy(bufs[k], acc_sh.at[idx_ring.at[k]], ssem).wait()

    for k in range(NBUF):
        fetch_start(k, k)

    def quad(g, _):
        for k in range(NBUF):
            c = NBUF * g + k
            fetch_wait(c, k)
            scat(c, k)

            def refill(c=c, k=k):
                fetch_start(c + NBUF, k)
            pl.when(c + NBUF <= NCHUNK - 1)(refill)
        return 0
    lax.fori_loop(0, (NCHUNK - 1) // NBUF, quad, 0)

    fetch_wait(NCHUNK - 1, 0)
    scat(NCHUNK - 1, 0)

    def drain(i, _):
        pltpu.make_async_copy(ones_v, cnt_sh.at[idx_ring.at[0]], csem).wait()
        return 0
    lax.fori_loop(0, NCHUNK, drain, 0)
    plsc.subcore_barrier()

    pltpu.sync_copy(acc_sh.at[pl.ds(base_r, RPT), :],
                    psum_hbm.at[cid, pl.ds(base_r, RPT), :])
    pltpu.sync_copy(cnt_sh.at[pl.ds(base_r, RPT)],
                    pcnt_hbm.at[pl.ds(cid * N_PAD + base_r, RPT)])
    plsc.subcore_barrier()

    @pl.when(sid == 0)
    def _():
        pltpu.sync_copy(ones_v.at[pl.ds(0, L)],
                        flag_hbm.at[pl.ds(cid * L, L)])
        fbuf[...] = jnp.zeros((L,), jnp.float32)

        def poll(i, found):
            def do_poll():
                pltpu.sync_copy(flag_hbm.at[pl.ds((1 - cid) * L, L)], fbuf)
            pl.when(found < 0.5)(do_poll)
            return jnp.maximum(found, fbuf[...][0])
        lax.fori_loop(0, 256, poll, jnp.float32(0.0))
    plsc.subcore_barrier()

    row0 = wid * R2
    ocid = 1 - cid
    pltpu.sync_copy(cnt_sh.at[pl.ds(row0, R2)], ca)
    pltpu.sync_copy(pcnt_hbm.at[pl.ds(ocid * N_PAD + row0, R2)], cb)
    for k in range(NBLK):
        rbase = row0 + k * B

        def block(rbase=rbase, k=k):
            pltpu.sync_copy(acc_sh.at[pl.ds(rbase, B), :], bufs[0])
            pltpu.sync_copy(psum_hbm.at[ocid, pl.ds(rbase, B), :], bufs[1])

            def group(g, _):
                i0 = k * B + g * L
                cv = ca[pl.ds(i0, L)] + cb[pl.ds(i0, L)]
                rcpb[pl.ds(g * L, L)] = 1.0 / jnp.maximum(cv, 1.0)
                return 0
            lax.fori_loop(0, B // L, group, 0)
            rcpb[pl.ds(B, L)] = jnp.ones((L,), jnp.float32)

            def row(r, _):
                s = rcpb[pl.ds(r, L)][0]

                def col(j, _):
                    sl = pl.ds(j * L, L)
                    bufs[0][r, sl] = (bufs[0][r, sl] + bufs[1][r, sl]) * s
                    return 0
                lax.fori_loop(0, D // L, col, 0)
                return 0
            lax.fori_loop(0, B, row, 0)
            pltpu.sync_copy(bufs[0], out_hbm.at[pl.ds(rbase, B), :])
        pl.when(rbase < N_NODES)(block)


def kernel(msg, index, t, dim_size):
    del t, dim_size
    idx32 = index.astype(jnp.int32)
    out, _, _, _ = _fused_stage(msg, idx32)
    return out


# speedup vs baseline: 1.0376x; 1.0214x over previous
import functools

import jax
import jax.numpy as jnp
from jax import lax
from jax.experimental import pallas as pl
from jax.experimental.pallas import tpu as pltpu
from jax.experimental.pallas import tpu_sc as plsc

N_EDGES = 320000
D = 128
N_NODES = 10000
N_PAD = 10240
NC = 2
NS = 16
L = 16
NW = NC * NS
EPT = N_EDGES // NW
B = 80
NCHUNK = EPT // B
NBUF = 4
RPT = N_PAD // NS
R2 = N_PAD // NW
NBLK = R2 // B

_mesh = plsc.VectorSubcoreMesh(core_axis_name="c", subcore_axis_name="s")


def _zero_vmem(ref, nwords):
    def body(j, _):
        ref[pl.ds(j * L, L)] = jnp.zeros((L,), jnp.float32)
        return 0
    lax.fori_loop(0, nwords // L, body, 0)


@functools.partial(
    pl.kernel,
    out_type=(
        jax.ShapeDtypeStruct((N_NODES, D), jnp.float32),
        jax.ShapeDtypeStruct((NC, N_PAD, D), jnp.float32),
        jax.ShapeDtypeStruct((NC * N_PAD,), jnp.float32),
        jax.ShapeDtypeStruct((NC * L,), jnp.float32),
    ),
    mesh=_mesh,
    scratch_types=[
        pltpu.VMEM_SHARED((N_PAD, D), jnp.float32),
        pltpu.VMEM_SHARED((N_PAD,), jnp.float32),
        pltpu.VMEM((NBUF, B), jnp.int32),
        pltpu.VMEM((B, D), jnp.float32),
        pltpu.VMEM((B, D), jnp.float32),
        pltpu.VMEM((B, D), jnp.float32),
        pltpu.VMEM((B, D), jnp.float32),
        pltpu.VMEM((B,), jnp.float32),
        pltpu.VMEM((RPT,), jnp.float32),
        pltpu.VMEM((R2,), jnp.float32),
        pltpu.VMEM((R2,), jnp.float32),
        pltpu.VMEM((B + L,), jnp.float32),
        pltpu.VMEM((L,), jnp.float32),
        pltpu.SemaphoreType.DMA,
        pltpu.SemaphoreType.DMA,
        pltpu.SemaphoreType.DMA,
        pltpu.SemaphoreType.DMA,
        pltpu.SemaphoreType.DMA,
        pltpu.SemaphoreType.DMA,
    ],
)
def _fused_stage(msg_hbm, idx_hbm, out_hbm, psum_hbm, pcnt_hbm, flag_hbm,
                 acc_sh, cnt_sh, idx_ring, rows0, rows1, rows2, rows3,
                 ones_v, zvec_v, ca, cb, rcpb, fbuf,
                 fsem0, fsem1, fsem2, fsem3, ssem, csem):
    cid = lax.axis_index("c")
    sid = lax.axis_index("s")
    wid = cid * NS + sid
    ebase = wid * EPT
    bufs = (rows0, rows1, rows2, rows3)
    fsems = (fsem0, fsem1, fsem2, fsem3)

    def zrow(r, _):
        def zcol(j, _):
            rows0[r, pl.ds(j * L, L)] = jnp.zeros((L,), jnp.float32)
            return 0
        lax.fori_loop(0, D // L, zcol, 0)
        return 0
    lax.fori_loop(0, B, zrow, 0)
    _zero_vmem(zvec_v, RPT)

    def one(j, _):
        ones_v[pl.ds(j * L, L)] = jnp.ones((L,), jnp.float32)
        return 0
    lax.fori_loop(0, B // L, one, 0)

    @pl.when(sid == 0)
    def _():
        pltpu.sync_copy(zvec_v.at[pl.ds(0, L)], flag_hbm.at[pl.ds(cid * L, L)])

    base_r = sid * RPT
    for k in range(RPT // B):
        pltpu.sync_copy(rows0, acc_sh.at[pl.ds(base_r + k * B, B), :])
    pltpu.sync_copy(zvec_v, cnt_sh.at[pl.ds(base_r, RPT)])
    plsc.subcore_barrier()

    def fetch_start(c, k):
        pltpu.async_copy(msg_hbm.at[pl.ds(ebase + c * B, B), :],
                         bufs[k], fsems[k])
        pltpu.async_copy(idx_hbm.at[pl.ds(ebase + c * B, B)],
                         idx_ring.at[k], fsems[k])

    def fetch_wait(c, k):
        pltpu.make_async_copy(msg_hbm.at[pl.ds(ebase + c * B, B), :],
                              bufs[k], fsems[k]).wait()
        pltpu.make_async_copy(idx_hbm.at[pl.ds(ebase + c * B, B)],
                              idx_ring.at[k], fsems[k]).wait()

    def scat(c, k):
        pltpu.async_copy(bufs[k], acc_sh.at[idx_ring.at[k]], ssem, add=True)
        pltpu.async_copy(ones_v, cnt_sh.at[idx_ring.at[k]], csem, add=True)
        pltpu.make_async_copy(bufs[k], acc_sh.at[idx_ring.at[k]], ssem).wait()

    for k in range(NBUF):
        fetch_start(k, k)

    def quad(g, _):
        for k in range(NBUF):
            c = NBUF * g + k
            fetch_wait(c, k)
            scat(c, k)

            def refill(c=c, k=k):
                fetch_start(c + NBUF, k)
            pl.when(c + NBUF <= NCHUNK - 1)(refill)
        return 0
    lax.fori_loop(0, (NCHUNK - 1) // NBUF, quad, 0)

    fetch_wait(NCHUNK - 1, 0)
    scat(NCHUNK - 1, 0)

    def drain(i, _):
        pltpu.make_async_copy(ones_v, cnt_sh.at[idx_ring.at[0]], csem).wait()
        return 0
    lax.fori_loop(0, NCHUNK, drain, 0)
    plsc.subcore_barrier()

    hbase = (1 - cid) * (N_PAD // NC) + sid * (N_PAD // NC // NS)
    pltpu.sync_copy(acc_sh.at[pl.ds(hbase, N_PAD // NC // NS), :],
                    psum_hbm.at[cid, pl.ds(hbase, N_PAD // NC // NS), :])
    pltpu.sync_copy(cnt_sh.at[pl.ds(base_r, RPT)],
                    pcnt_hbm.at[pl.ds(cid * N_PAD + base_r, RPT)])
    plsc.subcore_barrier()

    @pl.when(sid == 0)
    def _():
        pltpu.sync_copy(ones_v.at[pl.ds(0, L)],
                        flag_hbm.at[pl.ds(cid * L, L)])
        fbuf[...] = jnp.zeros((L,), jnp.float32)

        def poll(i, found):
            def do_poll():
                pltpu.sync_copy(flag_hbm.at[pl.ds((1 - cid) * L, L)], fbuf)
            pl.when(found < 0.5)(do_poll)
            return jnp.maximum(found, fbuf[...][0])
        lax.fori_loop(0, 256, poll, jnp.float32(0.0))
    plsc.subcore_barrier()

    row0 = wid * R2
    ocid = 1 - cid
    pltpu.sync_copy(cnt_sh.at[pl.ds(row0, R2)], ca)
    pltpu.sync_copy(pcnt_hbm.at[pl.ds(ocid * N_PAD + row0, R2)], cb)

    for k in range(NBLK):
        rbase = row0 + k * B

        def block(rbase=rbase, k=k, pair=0):
            pltpu.sync_copy(acc_sh.at[pl.ds(rbase, B), :], bufs[0])
            pltpu.sync_copy(psum_hbm.at[ocid, pl.ds(rbase, B), :], bufs[1])

            def group(g, _):
                i0 = k * B + g * L
                cv = ca[pl.ds(i0, L)] + cb[pl.ds(i0, L)]
                rcpb[pl.ds(g * L, L)] = 1.0 / jnp.maximum(cv, 1.0)
                return 0
            lax.fori_loop(0, B // L, group, 0)
            rcpb[pl.ds(B, L)] = jnp.ones((L,), jnp.float32)

            def row(r, _):
                s = rcpb[pl.ds(r, L)][0]

                def col(j, _):
                    sl = pl.ds(j * L, L)
                    bufs[2 * pair][r, sl] = (
                        bufs[2 * pair][r, sl] + bufs[2 * pair + 1][r, sl]) * s
                    return 0
                lax.fori_loop(0, D // L, col, 0)
                return 0
            lax.fori_loop(0, B, row, 0)
            pltpu.sync_copy(bufs[2 * pair], out_hbm.at[pl.ds(rbase, B), :])
        pl.when(rbase < N_NODES)(block)


def kernel(msg, index, t, dim_size):
    del t, dim_size
    idx32 = index.astype(jnp.int32)
    out, _, _, _ = _fused_stage(msg, idx32)
    return out


# speedup vs baseline: 1.0800x; 1.0409x over previous
import functools

import jax
import jax.numpy as jnp
from jax import lax
from jax.experimental import pallas as pl
from jax.experimental.pallas import tpu as pltpu
from jax.experimental.pallas import tpu_sc as plsc

N_EDGES = 320000
D = 128
N_NODES = 10000
N_PAD = 10240
NC = 2
NS = 16
L = 16
NW = NC * NS
EPT = N_EDGES // NW
B = 80
NCHUNK = EPT // B
NBUF = 4
RPT = N_PAD // NS
R2 = N_PAD // NW
NBLK = R2 // B

_mesh = plsc.VectorSubcoreMesh(core_axis_name="c", subcore_axis_name="s")


def _zero_vmem(ref, nwords):
    def body(j, _):
        ref[pl.ds(j * L, L)] = jnp.zeros((L,), jnp.float32)
        return 0
    lax.fori_loop(0, nwords // L, body, 0)


@functools.partial(
    pl.kernel,
    out_type=(
        jax.ShapeDtypeStruct((N_NODES, D), jnp.float32),
        jax.ShapeDtypeStruct((NC, N_PAD, D), jnp.float32),
        jax.ShapeDtypeStruct((NC * N_PAD,), jnp.float32),
        jax.ShapeDtypeStruct((NC * L,), jnp.float32),
    ),
    mesh=_mesh,
    scratch_types=[
        pltpu.VMEM_SHARED((N_PAD, D), jnp.float32),
        pltpu.VMEM_SHARED((N_PAD,), jnp.float32),
        pltpu.VMEM((NBUF, B), jnp.int32),
        pltpu.VMEM((B, D), jnp.float32),
        pltpu.VMEM((B, D), jnp.float32),
        pltpu.VMEM((B, D), jnp.float32),
        pltpu.VMEM((B, D), jnp.float32),
        pltpu.VMEM((B,), jnp.float32),
        pltpu.VMEM((RPT,), jnp.float32),
        pltpu.VMEM((R2,), jnp.float32),
        pltpu.VMEM((R2,), jnp.float32),
        pltpu.VMEM((B + L,), jnp.float32),
        pltpu.VMEM((L,), jnp.float32),
        pltpu.SemaphoreType.DMA,
        pltpu.SemaphoreType.DMA,
        pltpu.SemaphoreType.DMA,
        pltpu.SemaphoreType.DMA,
        pltpu.SemaphoreType.DMA,
        pltpu.SemaphoreType.DMA,
    ],
)
def _fused_stage(msg_hbm, idx_hbm, out_hbm, psum_hbm, pcnt_hbm, flag_hbm,
                 acc_sh, cnt_sh, idx_ring, rows0, rows1, rows2, rows3,
                 ones_v, zvec_v, ca, cb, rcpb, fbuf,
                 fsem0, fsem1, fsem2, fsem3, ssem, csem):
    cid = lax.axis_index("c")
    sid = lax.axis_index("s")
    wid = cid * NS + sid
    ebase = wid * EPT
    bufs = (rows0, rows1, rows2, rows3)
    fsems = (fsem0, fsem1, fsem2, fsem3)

    def zrow(r, _):
        def zcol(j, _):
            rows0[r, pl.ds(j * L, L)] = jnp.zeros((L,), jnp.float32)
            return 0
        lax.fori_loop(0, D // L, zcol, 0)
        return 0
    lax.fori_loop(0, B, zrow, 0)
    _zero_vmem(zvec_v, RPT)

    def one(j, _):
        ones_v[pl.ds(j * L, L)] = jnp.ones((L,), jnp.float32)
        return 0
    lax.fori_loop(0, B // L, one, 0)

    @pl.when(sid == 0)
    def _():
        pltpu.sync_copy(zvec_v.at[pl.ds(0, L)], flag_hbm.at[pl.ds(cid * L, L)])

    base_r = sid * RPT
    for k in range(RPT // B):
        pltpu.sync_copy(rows0, acc_sh.at[pl.ds(base_r + k * B, B), :])
    pltpu.sync_copy(zvec_v, cnt_sh.at[pl.ds(base_r, RPT)])
    plsc.subcore_barrier()

    def fetch_start(c, k):
        pltpu.async_copy(msg_hbm.at[pl.ds(ebase + c * B, B), :],
                         bufs[k], fsems[k])
        pltpu.async_copy(idx_hbm.at[pl.ds(ebase + c * B, B)],
                         idx_ring.at[k], fsems[k])

    def fetch_wait(c, k):
        pltpu.make_async_copy(msg_hbm.at[pl.ds(ebase + c * B, B), :],
                              bufs[k], fsems[k]).wait()
        pltpu.make_async_copy(idx_hbm.at[pl.ds(ebase + c * B, B)],
                              idx_ring.at[k], fsems[k]).wait()

    def scat(c, k):
        pltpu.async_copy(bufs[k], acc_sh.at[idx_ring.at[k]], ssem, add=True)
        pltpu.async_copy(ones_v, cnt_sh.at[idx_ring.at[k]], csem, add=True)
        pltpu.make_async_copy(bufs[k], acc_sh.at[idx_ring.at[k]], ssem).wait()

    for k in range(NBUF):
        fetch_start(k, k)

    def quad(g, _):
        for k in range(NBUF):
            c = NBUF * g + k
            fetch_wait(c, k)
            scat(c, k)

            def refill(c=c, k=k):
                fetch_start(c + NBUF, k)
            pl.when(c + NBUF <= NCHUNK - 1)(refill)
        return 0
    lax.fori_loop(0, (NCHUNK - 1) // NBUF, quad, 0)

    fetch_wait(NCHUNK - 1, 0)
    scat(NCHUNK - 1, 0)

    def drain(i, _):
        pltpu.make_async_copy(ones_v, cnt_sh.at[idx_ring.at[0]], csem).wait()
        return 0
    lax.fori_loop(0, NCHUNK, drain, 0)
    plsc.subcore_barrier()

    hbase = (1 - cid) * (N_PAD // NC) + sid * (N_PAD // NC // NS)
    pltpu.sync_copy(acc_sh.at[pl.ds(hbase, N_PAD // NC // NS), :],
                    psum_hbm.at[cid, pl.ds(hbase, N_PAD // NC // NS), :])
    pltpu.sync_copy(cnt_sh.at[pl.ds(base_r, RPT)],
                    pcnt_hbm.at[pl.ds(cid * N_PAD + base_r, RPT)])
    plsc.subcore_barrier()

    @pl.when(sid == 0)
    def _():
        pltpu.sync_copy(ones_v.at[pl.ds(0, L)],
                        flag_hbm.at[pl.ds(cid * L, L)])
        fbuf[...] = jnp.zeros((L,), jnp.float32)

        def poll(i, found):
            def do_poll():
                pltpu.sync_copy(flag_hbm.at[pl.ds((1 - cid) * L, L)], fbuf)
            pl.when(found < 0.5)(do_poll)
            return jnp.maximum(found, fbuf[...][0])
        lax.fori_loop(0, 256, poll, jnp.float32(0.0))
    plsc.subcore_barrier()

    row0 = wid * R2
    ocid = 1 - cid
    pltpu.sync_copy(cnt_sh.at[pl.ds(row0, R2)], ca)
    pltpu.sync_copy(pcnt_hbm.at[pl.ds(ocid * N_PAD + row0, R2)], cb)

    def rd_start(k, pair):
        rb = row0 + k * B
        pltpu.async_copy(acc_sh.at[pl.ds(rb, B), :],
                         bufs[2 * pair], fsems[2 + pair])
        pltpu.async_copy(psum_hbm.at[ocid, pl.ds(rb, B), :],
                         bufs[2 * pair + 1], fsems[pair])

    def rd_wait(k, pair):
        rb = row0 + k * B
        pltpu.make_async_copy(acc_sh.at[pl.ds(rb, B), :],
                              bufs[2 * pair], fsems[2 + pair]).wait()
        pltpu.make_async_copy(psum_hbm.at[ocid, pl.ds(rb, B), :],
                              bufs[2 * pair + 1], fsems[pair]).wait()

    pl.when(row0 < N_NODES)(lambda: rd_start(0, 0))
    for k in range(NBLK):
        rbase = row0 + k * B

        def block(rbase=rbase, k=k, pair=k % 2):
            rd_wait(k, pair)
            if k + 1 < NBLK:
                def pre():
                    rd_start(k + 1, 1 - pair)
                pl.when(row0 + (k + 1) * B < N_NODES)(pre)

            def group(g, _):
                i0 = k * B + g * L
                cv = ca[pl.ds(i0, L)] + cb[pl.ds(i0, L)]
                rcpb[pl.ds(g * L, L)] = 1.0 / jnp.maximum(cv, 1.0)
                return 0
            lax.fori_loop(0, B // L, group, 0)
            rcpb[pl.ds(B, L)] = jnp.ones((L,), jnp.float32)

            def row(r, _):
                s = rcpb[pl.ds(r, L)][0]

                def col(j, _):
                    sl = pl.ds(j * L, L)
                    bufs[2 * pair][r, sl] = (
                        bufs[2 * pair][r, sl] + bufs[2 * pair + 1][r, sl]) * s
                    return 0
                lax.fori_loop(0, D // L, col, 0)
                return 0
            lax.fori_loop(0, B, row, 0)
            pltpu.sync_copy(bufs[2 * pair], out_hbm.at[pl.ds(rbase, B), :])
        pl.when(rbase < N_NODES)(block)


def kernel(msg, index, t, dim_size):
    del t, dim_size
    idx32 = index.astype(jnp.int32)
    out, _, _, _ = _fused_stage(msg, idx32)
    return out


# speedup vs baseline: 1.0917x; 1.0108x over previous
import functools

import jax
import jax.numpy as jnp
from jax import lax
from jax.experimental import pallas as pl
from jax.experimental.pallas import tpu as pltpu
from jax.experimental.pallas import tpu_sc as plsc

N_EDGES = 320000
D = 128
N_NODES = 10000
N_PAD = 10240
NC = 2
NS = 16
L = 16
NW = NC * NS
EPT = N_EDGES // NW
B = 80
NCHUNK = EPT // B
NBUF = 4
RPT = N_PAD // NS
R2 = N_PAD // NW
NBLK = R2 // B

_mesh = plsc.VectorSubcoreMesh(core_axis_name="c", subcore_axis_name="s")


def _zero_vmem(ref, nwords):
    def body(j, _):
        ref[pl.ds(j * L, L)] = jnp.zeros((L,), jnp.float32)
        return 0
    lax.fori_loop(0, nwords // L, body, 0)


@functools.partial(
    pl.kernel,
    out_type=(
        jax.ShapeDtypeStruct((N_NODES, D), jnp.float32),
        jax.ShapeDtypeStruct((NC, N_PAD, D), jnp.float32),
        jax.ShapeDtypeStruct((NC * N_PAD,), jnp.float32),
        jax.ShapeDtypeStruct((NC * L,), jnp.float32),
    ),
    mesh=_mesh,
    scratch_types=[
        pltpu.VMEM_SHARED((N_PAD, D), jnp.float32),
        pltpu.VMEM_SHARED((N_PAD,), jnp.float32),
        pltpu.VMEM((NBUF, B), jnp.int32),
        pltpu.VMEM((B, D), jnp.float32),
        pltpu.VMEM((B, D), jnp.float32),
        pltpu.VMEM((B, D), jnp.float32),
        pltpu.VMEM((B, D), jnp.float32),
        pltpu.VMEM((B,), jnp.float32),
        pltpu.VMEM((RPT,), jnp.float32),
        pltpu.VMEM((R2,), jnp.float32),
        pltpu.VMEM((R2,), jnp.float32),
        pltpu.VMEM((B + L,), jnp.float32),
        pltpu.VMEM((L,), jnp.float32),
        pltpu.SemaphoreType.DMA,
        pltpu.SemaphoreType.DMA,
        pltpu.SemaphoreType.DMA,
        pltpu.SemaphoreType.DMA,
        pltpu.SemaphoreType.DMA,
        pltpu.SemaphoreType.DMA,
    ],
)
def _fused_stage(msg_hbm, idx_hbm, out_hbm, psum_hbm, pcnt_hbm, flag_hbm,
                 acc_sh, cnt_sh, idx_ring, rows0, rows1, rows2, rows3,
                 ones_v, zvec_v, ca, cb, rcpb, fbuf,
                 fsem0, fsem1, fsem2, fsem3, ssem, csem):
    cid = lax.axis_index("c")
    sid = lax.axis_index("s")
    wid = cid * NS + sid
    ebase = wid * EPT
    bufs = (rows0, rows1, rows2, rows3)
    fsems = (fsem0, fsem1, fsem2, fsem3)

    def zrow(r, _):
        def zcol(j, _):
            rows0[r, pl.ds(j * L, L)] = jnp.zeros((L,), jnp.float32)
            return 0
        lax.fori_loop(0, D // L, zcol, 0)
        return 0
    lax.fori_loop(0, B, zrow, 0)
    _zero_vmem(zvec_v, RPT)

    def one(j, _):
        ones_v[pl.ds(j * L, L)] = jnp.ones((L,), jnp.float32)
        return 0
    lax.fori_loop(0, B // L, one, 0)

    @pl.when(sid == 0)
    def _():
        pltpu.sync_copy(zvec_v.at[pl.ds(0, L)], flag_hbm.at[pl.ds(cid * L, L)])

    base_r = sid * RPT

    def fetch_start(c, k):
        pltpu.async_copy(msg_hbm.at[pl.ds(ebase + c * B, B), :],
                         bufs[k], fsems[k])
        pltpu.async_copy(idx_hbm.at[pl.ds(ebase + c * B, B)],
                         idx_ring.at[k], fsems[k])

    def fetch_wait(c, k):
        pltpu.make_async_copy(msg_hbm.at[pl.ds(ebase + c * B, B), :],
                              bufs[k], fsems[k]).wait()
        pltpu.make_async_copy(idx_hbm.at[pl.ds(ebase + c * B, B)],
                              idx_ring.at[k], fsems[k]).wait()

    def scat(c, k):
        pltpu.async_copy(bufs[k], acc_sh.at[idx_ring.at[k]], ssem, add=True)
        pltpu.async_copy(ones_v, cnt_sh.at[idx_ring.at[k]], csem, add=True)
        pltpu.make_async_copy(bufs[k], acc_sh.at[idx_ring.at[k]], ssem).wait()

    for k in range(1, NBUF):
        fetch_start(k, k)
    for k in range(RPT // B):
        pltpu.sync_copy(rows0, acc_sh.at[pl.ds(base_r + k * B, B), :])
    pltpu.sync_copy(zvec_v, cnt_sh.at[pl.ds(base_r, RPT)])
    fetch_start(0, 0)
    plsc.subcore_barrier()

    def quad(g, _):
        for k in range(NBUF):
            c = NBUF * g + k
            fetch_wait(c, k)
            scat(c, k)

            def refill(c=c, k=k):
                fetch_start(c + NBUF, k)
            pl.when(c + NBUF <= NCHUNK - 1)(refill)
        return 0
    lax.fori_loop(0, (NCHUNK - 1) // NBUF, quad, 0)

    fetch_wait(NCHUNK - 1, 0)
    scat(NCHUNK - 1, 0)

    def drain(i, _):
        pltpu.make_async_copy(ones_v, cnt_sh.at[idx_ring.at[0]], csem).wait()
        return 0
    lax.fori_loop(0, NCHUNK, drain, 0)
    plsc.subcore_barrier()

    hbase = (1 - cid) * (N_PAD // NC) + sid * (N_PAD // NC // NS)
    pltpu.sync_copy(acc_sh.at[pl.ds(hbase, N_PAD // NC // NS), :],
                    psum_hbm.at[cid, pl.ds(hbase, N_PAD // NC // NS), :])
    pltpu.sync_copy(cnt_sh.at[pl.ds(base_r, RPT)],
                    pcnt_hbm.at[pl.ds(cid * N_PAD + base_r, RPT)])
    plsc.subcore_barrier()

    @pl.when(sid == 0)
    def _():
        pltpu.sync_copy(ones_v.at[pl.ds(0, L)],
                        flag_hbm.at[pl.ds(cid * L, L)])
        fbuf[...] = jnp.zeros((L,), jnp.float32)

        def poll(i, found):
            def do_poll():
                pltpu.sync_copy(flag_hbm.at[pl.ds((1 - cid) * L, L)], fbuf)
            pl.when(found < 0.5)(do_poll)
            return jnp.maximum(found, fbuf[...][0])
        lax.fori_loop(0, 256, poll, jnp.float32(0.0))
    plsc.subcore_barrier()

    row0 = wid * R2
    ocid = 1 - cid
    pltpu.sync_copy(cnt_sh.at[pl.ds(row0, R2)], ca)
    pltpu.sync_copy(pcnt_hbm.at[pl.ds(ocid * N_PAD + row0, R2)], cb)

    def rd_start(k, pair):
        rb = row0 + k * B
        pltpu.async_copy(acc_sh.at[pl.ds(rb, B), :],
                         bufs[2 * pair], fsems[2 + pair])
        pltpu.async_copy(psum_hbm.at[ocid, pl.ds(rb, B), :],
                         bufs[2 * pair + 1], fsems[pair])

    def rd_wait(k, pair):
        rb = row0 + k * B
        pltpu.make_async_copy(acc_sh.at[pl.ds(rb, B), :],
                              bufs[2 * pair], fsems[2 + pair]).wait()
        pltpu.make_async_copy(psum_hbm.at[ocid, pl.ds(rb, B), :],
                              bufs[2 * pair + 1], fsems[pair]).wait()

    pl.when(row0 < N_NODES)(lambda: rd_start(0, 0))
    for k in range(NBLK):
        rbase = row0 + k * B

        def block(rbase=rbase, k=k, pair=k % 2):
            rd_wait(k, pair)
            if k + 1 < NBLK:
                def pre():
                    rd_start(k + 1, 1 - pair)
                pl.when(row0 + (k + 1) * B < N_NODES)(pre)

            def group(g, _):
                i0 = k * B + g * L
                cv = ca[pl.ds(i0, L)] + cb[pl.ds(i0, L)]
                rcpb[pl.ds(g * L, L)] = 1.0 / jnp.maximum(cv, 1.0)
                return 0
            lax.fori_loop(0, B // L, group, 0)
            rcpb[pl.ds(B, L)] = jnp.ones((L,), jnp.float32)

            def row(r, _):
                s = rcpb[pl.ds(r, L)][0]
                for j in range(D // L):
                    sl = pl.ds(j * L, L)
                    bufs[2 * pair][r, sl] = (
                        bufs[2 * pair][r, sl] + bufs[2 * pair + 1][r, sl]) * s
                return 0
            lax.fori_loop(0, B, row, 0)
            pltpu.sync_copy(bufs[2 * pair], out_hbm.at[pl.ds(rbase, B), :])
        pl.when(rbase < N_NODES)(block)


def kernel(msg, index, t, dim_size):
    del t, dim_size
    idx32 = index.astype(jnp.int32)
    out, _, _, _ = _fused_stage(msg, idx32)
    return out


# speedup vs baseline: 1.1226x; 1.0283x over previous
import functools

import jax
import jax.numpy as jnp
from jax import lax
from jax.experimental import pallas as pl
from jax.experimental.pallas import tpu as pltpu
from jax.experimental.pallas import tpu_sc as plsc

N_EDGES = 320000
D = 128
N_NODES = 10000
N_PAD = 10240
NC = 2
NS = 16
L = 16
NW = NC * NS
EPT = N_EDGES // NW
B = 80
NCHUNK = EPT // B
NBUF = 4
RPT = N_PAD // NS
R2 = N_PAD // NW
NBLK = R2 // B

_mesh = plsc.VectorSubcoreMesh(core_axis_name="c", subcore_axis_name="s")


def _zero_vmem(ref, nwords):
    def body(j, _):
        ref[pl.ds(j * L, L)] = jnp.zeros((L,), jnp.float32)
        return 0
    lax.fori_loop(0, nwords // L, body, 0)


@functools.partial(
    pl.kernel,
    out_type=(
        jax.ShapeDtypeStruct((N_NODES, D), jnp.float32),
        jax.ShapeDtypeStruct((NC, N_PAD, D), jnp.float32),
        jax.ShapeDtypeStruct((NC * N_PAD,), jnp.float32),
        jax.ShapeDtypeStruct((NC * L,), jnp.float32),
    ),
    mesh=_mesh,
    scratch_types=[
        pltpu.VMEM_SHARED((N_PAD, D), jnp.float32),
        pltpu.VMEM_SHARED((N_PAD,), jnp.float32),
        pltpu.VMEM((NBUF, B), jnp.int32),
        pltpu.VMEM((B, D), jnp.float32),
        pltpu.VMEM((B, D), jnp.float32),
        pltpu.VMEM((B, D), jnp.float32),
        pltpu.VMEM((B, D), jnp.float32),
        pltpu.VMEM((B,), jnp.float32),
        pltpu.VMEM((RPT,), jnp.float32),
        pltpu.VMEM((R2,), jnp.float32),
        pltpu.VMEM((R2,), jnp.float32),
        pltpu.VMEM((B + L,), jnp.float32),
        pltpu.VMEM((L,), jnp.float32),
        pltpu.SemaphoreType.DMA,
        pltpu.SemaphoreType.DMA,
        pltpu.SemaphoreType.DMA,
        pltpu.SemaphoreType.DMA,
        pltpu.SemaphoreType.DMA,
        pltpu.SemaphoreType.DMA,
    ],
)
def _fused_stage(msg_hbm, idx_hbm, out_hbm, psum_hbm, pcnt_hbm, flag_hbm,
                 acc_sh, cnt_sh, idx_ring, rows0, rows1, rows2, rows3,
                 ones_v, zvec_v, ca, cb, rcpb, fbuf,
                 fsem0, fsem1, fsem2, fsem3, ssem, csem):
    cid = lax.axis_index("c")
    sid = lax.axis_index("s")
    wid = cid * NS + sid
    ebase = wid * EPT
    bufs = (rows0, rows1, rows2, rows3)
    fsems = (fsem0, fsem1, fsem2, fsem3)

    def zrow(r, _):
        def zcol(j, _):
            rows0[r, pl.ds(j * L, L)] = jnp.zeros((L,), jnp.float32)
            return 0
        lax.fori_loop(0, D // L, zcol, 0)
        return 0
    lax.fori_loop(0, B, zrow, 0)
    _zero_vmem(zvec_v, RPT)

    def one(j, _):
        ones_v[pl.ds(j * L, L)] = jnp.ones((L,), jnp.float32)
        return 0
    lax.fori_loop(0, B // L, one, 0)

    @pl.when(sid == 0)
    def _():
        pltpu.sync_copy(zvec_v.at[pl.ds(0, L)], flag_hbm.at[pl.ds(cid * L, L)])

    base_r = sid * RPT

    def fetch_start(c, k):
        pltpu.async_copy(msg_hbm.at[pl.ds(ebase + c * B, B), :],
                         bufs[k], fsems[k])
        pltpu.async_copy(idx_hbm.at[pl.ds(ebase + c * B, B)],
                         idx_ring.at[k], fsems[k])

    def fetch_wait(c, k):
        pltpu.make_async_copy(msg_hbm.at[pl.ds(ebase + c * B, B), :],
                              bufs[k], fsems[k]).wait()
        pltpu.make_async_copy(idx_hbm.at[pl.ds(ebase + c * B, B)],
                              idx_ring.at[k], fsems[k]).wait()

    def scat(c, k):
        pltpu.async_copy(bufs[k], acc_sh.at[idx_ring.at[k]], ssem, add=True)
        pltpu.async_copy(ones_v, cnt_sh.at[idx_ring.at[k]], csem, add=True)
        pltpu.make_async_copy(bufs[k], acc_sh.at[idx_ring.at[k]], ssem).wait()

    for k in range(1, NBUF):
        fetch_start(k, k)
    for k in range(RPT // B):
        pltpu.sync_copy(rows0, acc_sh.at[pl.ds(base_r + k * B, B), :])
    pltpu.sync_copy(zvec_v, cnt_sh.at[pl.ds(base_r, RPT)])
    fetch_start(0, 0)
    plsc.subcore_barrier()

    def quad(g, _):
        for k in range(NBUF):
            c = NBUF * g + k
            fetch_wait(c, k)
            scat(c, k)

            def refill(c=c, k=k):
                fetch_start(c + NBUF, k)
            pl.when(c + NBUF <= NCHUNK - 1)(refill)
        return 0
    lax.fori_loop(0, (NCHUNK - 1) // NBUF, quad, 0)

    fetch_wait(NCHUNK - 1, 0)
    scat(NCHUNK - 1, 0)

    def drain(i, _):
        pltpu.make_async_copy(ones_v, cnt_sh.at[idx_ring.at[0]], csem).wait()
        return 0
    lax.fori_loop(0, NCHUNK, drain, 0)
    plsc.subcore_barrier()

    hbase = (1 - cid) * (N_PAD // NC) + sid * (N_PAD // NC // NS)
    pltpu.sync_copy(acc_sh.at[pl.ds(hbase, N_PAD // NC // NS), :],
                    psum_hbm.at[cid, pl.ds(hbase, N_PAD // NC // NS), :])
    pltpu.sync_copy(cnt_sh.at[pl.ds(base_r, RPT)],
                    pcnt_hbm.at[pl.ds(cid * N_PAD + base_r, RPT)])
    plsc.subcore_barrier()

    @pl.when(sid == 0)
    def _():
        pltpu.sync_copy(ones_v.at[pl.ds(0, L)],
                        flag_hbm.at[pl.ds(cid * L, L)])
        fbuf[...] = jnp.zeros((L,), jnp.float32)

        def poll(i, found):
            def do_poll():
                pltpu.sync_copy(flag_hbm.at[pl.ds((1 - cid) * L, L)], fbuf)
            pl.when(found < 0.5)(do_poll)
            return jnp.maximum(found, fbuf[...][0])
        lax.fori_loop(0, 64, poll, jnp.float32(0.0))
    plsc.subcore_barrier()

    row0 = wid * R2
    ocid = 1 - cid
    pltpu.sync_copy(cnt_sh.at[pl.ds(row0, R2)], ca)
    pltpu.sync_copy(pcnt_hbm.at[pl.ds(ocid * N_PAD + row0, R2)], cb)

    def rd_start(k, pair):
        rb = row0 + k * B
        pltpu.async_copy(acc_sh.at[pl.ds(rb, B), :],
                         bufs[2 * pair], fsems[2 + pair])
        pltpu.async_copy(psum_hbm.at[ocid, pl.ds(rb, B), :],
                         bufs[2 * pair + 1], fsems[pair])

    def rd_wait(k, pair):
        rb = row0 + k * B
        pltpu.make_async_copy(acc_sh.at[pl.ds(rb, B), :],
                              bufs[2 * pair], fsems[2 + pair]).wait()
        pltpu.make_async_copy(psum_hbm.at[ocid, pl.ds(rb, B), :],
                              bufs[2 * pair + 1], fsems[pair]).wait()

    pl.when(row0 < N_NODES)(lambda: rd_start(0, 0))
    for k in range(NBLK):
        rbase = row0 + k * B

        def block(rbase=rbase, k=k, pair=k % 2):
            rd_wait(k, pair)
            if k + 1 < NBLK:
                def pre():
                    rd_start(k + 1, 1 - pair)
                pl.when(row0 + (k + 1) * B < N_NODES)(pre)

            def group(g, _):
                i0 = k * B + g * L
                cv = ca[pl.ds(i0, L)] + cb[pl.ds(i0, L)]
                rcpb[pl.ds(g * L, L)] = 1.0 / jnp.maximum(cv, 1.0)
                return 0
            lax.fori_loop(0, B // L, group, 0)
            rcpb[pl.ds(B, L)] = jnp.ones((L,), jnp.float32)

            def row(r, _):
                s = rcpb[pl.ds(r, L)][0]
                for j in range(D // L):
                    sl = pl.ds(j * L, L)
                    bufs[2 * pair][r, sl] = (
                        bufs[2 * pair][r, sl] + bufs[2 * pair + 1][r, sl]) * s
                return 0
            lax.fori_loop(0, B, row, 0)
            pltpu.sync_copy(bufs[2 * pair], out_hbm.at[pl.ds(rbase, B), :])
        pl.when(rbase < N_NODES)(block)


def kernel(msg, index, t, dim_size):
    del t, dim_size
    idx32 = index.astype(jnp.int32)
    out, _, _, _ = _fused_stage(msg, idx32)
    return out
